# baseline (device time: 245837 ns/iter reference)
import jax
import jax.numpy as jnp
from jax import lax
from jax.experimental import pallas as pl
from jax.experimental.pallas import tpu as pltpu

B, SQ, H, D = 4, 32, 8, 128
HD = H * D
BH = B * H
SCALE = D ** -0.5
CHUNK = 512
NEG_INF = -1e30


def kernel(Q, K, V):
    b, sq, h, d = Q.shape
    skv = K.shape[1]
    assert (b, sq, h, d) == (B, SQ, H, D), Q.shape
    assert skv % CHUNK == 0, skv
    nc = skv // CHUNK

    def body(q_ref, k_ref, v_ref, o_ref,
             acc_ref, m_ref, l_ref,
             z_send, z_recv, L_send, L_recv,
             send_sems, recv_sems):
        step = pl.program_id(0)
        my_x = lax.axis_index("x")
        my_y = lax.axis_index("y")
        nbr = (1 - my_x, my_y)

        @pl.when(step == 0)
        def _init():
            bsem = pltpu.get_barrier_semaphore()
            pl.semaphore_signal(bsem, inc=1, device_id=nbr,
                                device_id_type=pl.DeviceIdType.MESH)
            pl.semaphore_wait(bsem, 1)
            acc_ref[...] = jnp.zeros_like(acc_ref)
            m_ref[...] = jnp.full_like(m_ref, NEG_INF)
            l_ref[...] = jnp.zeros_like(l_ref)

        for bi in range(B):
            for hi in range(H):
                col = bi * H + hi
                lo, hh = hi * D, (hi + 1) * D
                q = q_ref[bi, :, lo:hh].astype(jnp.bfloat16)
                k = k_ref[bi, :, lo:hh].astype(jnp.bfloat16)
                v = v_ref[bi, :, lo:hh].astype(jnp.bfloat16)
                s = lax.dot_general(
                    q, k, (((1,), (1,)), ((), ())),
                    preferred_element_type=jnp.float32) * SCALE
                m_prev = m_ref[:, col:col + 1]
                l_prev = l_ref[:, col:col + 1]
                m_cur = jnp.maximum(m_prev, jnp.max(s, axis=1, keepdims=True))
                p = jnp.exp(s - m_cur)
                corr = jnp.exp(m_prev - m_cur)
                l_new = l_prev * corr + jnp.sum(p, axis=1, keepdims=True)
                pv = lax.dot_general(
                    p.astype(jnp.bfloat16), v, (((1,), (0,)), ((), ())),
                    preferred_element_type=jnp.float32)
                acc_ref[bi, :, lo:hh] = acc_ref[bi, :, lo:hh] * corr + pv
                m_ref[:, col:col + 1] = m_cur
                l_ref[:, col:col + 1] = l_new

        @pl.when(step == nc - 1)
        def _finish():
            for bi in range(B):
                for hi in range(H):
                    col = bi * H + hi
                    lo, hh = hi * D, (hi + 1) * D
                    z_send[bi, :, lo:hh] = (
                        acc_ref[bi, :, lo:hh] / l_ref[:, col:col + 1]
                    ).astype(jnp.bfloat16)
            L_send[...] = m_ref[...] + jnp.log(l_ref[...])

            rdma_z = pltpu.make_async_remote_copy(
                src_ref=z_send, dst_ref=z_recv,
                send_sem=send_sems.at[0], recv_sem=recv_sems.at[0],
                device_id=nbr, device_id_type=pl.DeviceIdType.MESH)
            rdma_l = pltpu.make_async_remote_copy(
                src_ref=L_send, dst_ref=L_recv,
                send_sem=send_sems.at[1], recv_sem=recv_sems.at[1],
                device_id=nbr, device_id_type=pl.DeviceIdType.MESH)
            rdma_z.start()
            rdma_l.start()
            rdma_z.wait()
            rdma_l.wait()

            for bi in range(B):
                for hi in range(H):
                    col = bi * H + hi
                    lo, hh = hi * D, (hi + 1) * D
                    L_s = L_send[:, col:col + 1]
                    L_o = L_recv[:, col:col + 1]
                    m_t = jnp.maximum(L_s, L_o)
                    w_s = jnp.exp(L_s - m_t)
                    w_o = jnp.exp(L_o - m_t)
                    z_s = z_send[bi, :, lo:hh].astype(jnp.float32)
                    z_o = z_recv[bi, :, lo:hh].astype(jnp.float32)
                    o_ref[bi, :, lo:hh] = (z_s * w_s + z_o * w_o) / (w_s + w_o)

    out = pl.pallas_call(
        body,
        grid=(nc,),
        in_specs=[
            pl.BlockSpec((B, SQ, HD), lambda i: (0, 0, 0)),
            pl.BlockSpec((B, CHUNK, HD), lambda i: (0, i, 0)),
            pl.BlockSpec((B, CHUNK, HD), lambda i: (0, i, 0)),
        ],
        out_specs=pl.BlockSpec((B, SQ, HD), lambda i: (0, 0, 0)),
        out_shape=jax.ShapeDtypeStruct((B, SQ, HD), jnp.float32),
        scratch_shapes=[
            pltpu.VMEM((B, SQ, HD), jnp.float32),
            pltpu.VMEM((SQ, BH), jnp.float32),
            pltpu.VMEM((SQ, BH), jnp.float32),
            pltpu.VMEM((B, SQ, HD), jnp.bfloat16),
            pltpu.VMEM((B, SQ, HD), jnp.bfloat16),
            pltpu.VMEM((SQ, BH), jnp.float32),
            pltpu.VMEM((SQ, BH), jnp.float32),
            pltpu.SemaphoreType.DMA((2,)),
            pltpu.SemaphoreType.DMA((2,)),
        ],
        compiler_params=pltpu.CompilerParams(
            dimension_semantics=("arbitrary",),
            collective_id=0,
            vmem_limit_bytes=64 * 1024 * 1024,
        ),
    )(Q.reshape(B, SQ, HD), K.reshape(B, skv, HD), V.reshape(B, skv, HD))
    return out.reshape(B, SQ, H, D)


# device time: 198831 ns/iter; 1.2364x vs baseline; 1.2364x over previous
import jax
import jax.numpy as jnp
from jax import lax
from jax.experimental import pallas as pl
from jax.experimental.pallas import tpu as pltpu

B, SQ, H, D = 4, 32, 8, 128
HD = H * D
SCALE = D ** -0.5
CHUNK = 512
NEG_INF = -1e30


def kernel(Q, K, V):
    b, sq, h, d = Q.shape
    skv = K.shape[1]
    assert (b, sq, h, d) == (B, SQ, H, D), Q.shape
    assert skv % CHUNK == 0, skv
    nc = skv // CHUNK

    def body(q_ref, k_ref, v_ref, o_ref,
             acc_ref, m_ref, l_ref,
             z_send, z_recv, L_send, L_recv,
             send_sems, recv_sems):
        step = pl.program_id(0)
        my_x = lax.axis_index("x")
        my_y = lax.axis_index("y")
        nbr = (1 - my_x, my_y)

        @pl.when(step == 0)
        def _init():
            bsem = pltpu.get_barrier_semaphore()
            pl.semaphore_signal(bsem, inc=1, device_id=nbr,
                                device_id_type=pl.DeviceIdType.MESH)
            pl.semaphore_wait(bsem, 1)
            acc_ref[...] = jnp.zeros_like(acc_ref)
            m_ref[...] = jnp.full_like(m_ref, NEG_INF)
            l_ref[...] = jnp.zeros_like(l_ref)

        for hi in range(H):
            lo, hh = hi * D, (hi + 1) * D
            q = (q_ref[:, :, lo:hh] * SCALE).astype(jnp.bfloat16)
            k = k_ref[:, :, lo:hh].astype(jnp.bfloat16)
            v = v_ref[:, :, lo:hh].astype(jnp.bfloat16)
            s = lax.dot_general(
                q, k, (((2,), (2,)), ((0,), (0,))),
                preferred_element_type=jnp.float32)
            m_prev = m_ref[:, :, hi:hi + 1]
            l_prev = l_ref[:, :, hi:hi + 1]
            m_cur = jnp.maximum(m_prev, jnp.max(s, axis=2, keepdims=True))
            p = jnp.exp(s - m_cur)
            corr = jnp.exp(m_prev - m_cur)
            l_new = l_prev * corr + jnp.sum(p, axis=2, keepdims=True)
            pv = lax.dot_general(
                p.astype(jnp.bfloat16), v, (((2,), (1,)), ((0,), (0,))),
                preferred_element_type=jnp.float32)
            acc_ref[:, :, lo:hh] = acc_ref[:, :, lo:hh] * corr + pv
            m_ref[:, :, hi:hi + 1] = m_cur
            l_ref[:, :, hi:hi + 1] = l_new

        @pl.when(step == nc - 1)
        def _finish():
            for hi in range(H):
                lo, hh = hi * D, (hi + 1) * D
                z_send[:, :, lo:hh] = (
                    acc_ref[:, :, lo:hh] / l_ref[:, :, hi:hi + 1]
                ).astype(jnp.bfloat16)
            L_send[...] = m_ref[...] + jnp.log(l_ref[...])

            rdma_z = pltpu.make_async_remote_copy(
                src_ref=z_send, dst_ref=z_recv,
                send_sem=send_sems.at[0], recv_sem=recv_sems.at[0],
                device_id=nbr, device_id_type=pl.DeviceIdType.MESH)
            rdma_l = pltpu.make_async_remote_copy(
                src_ref=L_send, dst_ref=L_recv,
                send_sem=send_sems.at[1], recv_sem=recv_sems.at[1],
                device_id=nbr, device_id_type=pl.DeviceIdType.MESH)
            rdma_z.start()
            rdma_l.start()
            rdma_z.wait()
            rdma_l.wait()

            for hi in range(H):
                lo, hh = hi * D, (hi + 1) * D
                L_s = L_send[:, :, hi:hi + 1]
                L_o = L_recv[:, :, hi:hi + 1]
                m_t = jnp.maximum(L_s, L_o)
                w_s = jnp.exp(L_s - m_t)
                w_o = jnp.exp(L_o - m_t)
                z_s = z_send[:, :, lo:hh].astype(jnp.float32)
                z_o = z_recv[:, :, lo:hh].astype(jnp.float32)
                o_ref[:, :, lo:hh] = (z_s * w_s + z_o * w_o) / (w_s + w_o)

    out = pl.pallas_call(
        body,
        grid=(nc,),
        in_specs=[
            pl.BlockSpec((B, SQ, HD), lambda i: (0, 0, 0)),
            pl.BlockSpec((B, CHUNK, HD), lambda i: (0, i, 0)),
            pl.BlockSpec((B, CHUNK, HD), lambda i: (0, i, 0)),
        ],
        out_specs=pl.BlockSpec((B, SQ, HD), lambda i: (0, 0, 0)),
        out_shape=jax.ShapeDtypeStruct((B, SQ, HD), jnp.float32),
        scratch_shapes=[
            pltpu.VMEM((B, SQ, HD), jnp.float32),
            pltpu.VMEM((B, SQ, H), jnp.float32),
            pltpu.VMEM((B, SQ, H), jnp.float32),
            pltpu.VMEM((B, SQ, HD), jnp.bfloat16),
            pltpu.VMEM((B, SQ, HD), jnp.bfloat16),
            pltpu.VMEM((B, SQ, H), jnp.float32),
            pltpu.VMEM((B, SQ, H), jnp.float32),
            pltpu.SemaphoreType.DMA((2,)),
            pltpu.SemaphoreType.DMA((2,)),
        ],
        compiler_params=pltpu.CompilerParams(
            dimension_semantics=("arbitrary",),
            collective_id=0,
            vmem_limit_bytes=64 * 1024 * 1024,
        ),
    )(Q.reshape(B, SQ, HD), K.reshape(B, skv, HD), V.reshape(B, skv, HD))
    return out.reshape(B, SQ, H, D)


# device time: 90576 ns/iter; 2.7142x vs baseline; 2.1952x over previous
import jax
import jax.numpy as jnp
from jax import lax
from jax.experimental import pallas as pl
from jax.experimental.pallas import tpu as pltpu

B, SQ, H, D = 4, 32, 8, 128
HD = H * D
SCALE = D ** -0.5
CHUNK = 512
NEG_INF = -1e30


def _combine(z_a, L_a, z_b, L_b):
    m_t = jnp.maximum(L_a, L_b)
    w_a = jnp.exp(L_a - m_t)
    w_b = jnp.exp(L_b - m_t)
    denom = w_a + w_b
    z = (z_a * w_a + z_b * w_b) / denom
    return z, m_t + jnp.log(denom)


def kernel(Q, K, V):
    b, sq, h, d = Q.shape
    skv = K.shape[1]
    assert (b, sq, h, d) == (B, SQ, H, D), Q.shape
    half = skv // 2
    assert half % CHUNK == 0, skv
    nch = half // CHUNK

    def body(y_sc, q_ref, k_ref, v_ref, o_ref,
             kst, vst, cp_sems,
             acc_ref, m_ref, l_ref,
             z1s, z1r, L1s, L1r,
             z2s, z2r, L2s, L2r,
             send_sems, recv_sems):
        step = pl.program_id(0)
        my_x = lax.axis_index("x")
        my_y = lax.axis_index("y")
        nbr_y = (my_x, 1 - my_y)
        nbr_x = (1 - my_x, my_y)

        @pl.when(step == 0)
        def _init():
            bsem = pltpu.get_barrier_semaphore()
            for nbr in (nbr_y, nbr_x):
                pl.semaphore_signal(bsem, inc=1, device_id=nbr,
                                    device_id_type=pl.DeviceIdType.MESH)
            pl.semaphore_wait(bsem, 2)
            acc_ref[...] = jnp.zeros_like(acc_ref)
            m_ref[...] = jnp.full_like(m_ref, NEG_INF)
            l_ref[...] = jnp.zeros_like(l_ref)

        def head_copies(hi):
            ck = pltpu.make_async_copy(
                k_ref.at[:, :, hi, :], kst.at[hi], cp_sems.at[0, hi])
            cv = pltpu.make_async_copy(
                v_ref.at[:, :, hi, :], vst.at[hi], cp_sems.at[1, hi])
            return ck, cv

        for hi in range(H):
            for c in head_copies(hi):
                c.start()
        for hi in range(H):
            for c in head_copies(hi):
                c.wait()

            lo, hh = hi * D, (hi + 1) * D
            q = q_ref[:, :, lo:hh] * SCALE
            k = kst[hi]
            v = vst[hi]
            s = lax.dot_general(
                q, k, (((2,), (2,)), ((0,), (0,))),
                preferred_element_type=jnp.float32)
            m_prev = m_ref[:, :, hi:hi + 1]
            l_prev = l_ref[:, :, hi:hi + 1]
            m_cur = jnp.maximum(m_prev, jnp.max(s, axis=2, keepdims=True))
            p = jnp.exp(s - m_cur)
            corr = jnp.exp(m_prev - m_cur)
            l_new = l_prev * corr + jnp.sum(p, axis=2, keepdims=True)
            pv = lax.dot_general(
                p, v, (((2,), (1,)), ((0,), (0,))),
                preferred_element_type=jnp.float32)
            acc_ref[:, :, lo:hh] = acc_ref[:, :, lo:hh] * corr + pv
            m_ref[:, :, hi:hi + 1] = m_cur
            l_ref[:, :, hi:hi + 1] = l_new

        @pl.when(step == nch - 1)
        def _finish():
            for hi in range(H):
                lo, hh = hi * D, (hi + 1) * D
                z1s[:, :, lo:hh] = (
                    acc_ref[:, :, lo:hh] / l_ref[:, :, hi:hi + 1]
                ).astype(jnp.bfloat16)
            L1s[...] = m_ref[...] + jnp.log(l_ref[...])

            def exchange(zs, zr, Ls, Lr, nbr, si):
                rz = pltpu.make_async_remote_copy(
                    src_ref=zs, dst_ref=zr,
                    send_sem=send_sems.at[si], recv_sem=recv_sems.at[si],
                    device_id=nbr, device_id_type=pl.DeviceIdType.MESH)
                rl = pltpu.make_async_remote_copy(
                    src_ref=Ls, dst_ref=Lr,
                    send_sem=send_sems.at[si + 1],
                    recv_sem=recv_sems.at[si + 1],
                    device_id=nbr, device_id_type=pl.DeviceIdType.MESH)
                rz.start()
                rl.start()
                rz.wait()
                rl.wait()

            exchange(z1s, z1r, L1s, L1r, nbr_y, 0)
            for hi in range(H):
                lo, hh = hi * D, (hi + 1) * D
                z, L = _combine(
                    z1s[:, :, lo:hh].astype(jnp.float32),
                    L1s[:, :, hi:hi + 1],
                    z1r[:, :, lo:hh].astype(jnp.float32),
                    L1r[:, :, hi:hi + 1])
                z2s[:, :, lo:hh] = z.astype(jnp.bfloat16)
                L2s[:, :, hi:hi + 1] = L

            exchange(z2s, z2r, L2s, L2r, nbr_x, 2)
            for hi in range(H):
                lo, hh = hi * D, (hi + 1) * D
                z, _ = _combine(
                    z2s[:, :, lo:hh].astype(jnp.float32),
                    L2s[:, :, hi:hi + 1],
                    z2r[:, :, lo:hh].astype(jnp.float32),
                    L2r[:, :, hi:hi + 1])
                o_ref[:, :, lo:hh] = z

    grid_spec = pltpu.PrefetchScalarGridSpec(
        num_scalar_prefetch=1,
        grid=(nch,),
        in_specs=[
            pl.BlockSpec((B, SQ, HD), lambda i, y: (0, 0, 0)),
            pl.BlockSpec((B, CHUNK, H, D), lambda i, y: (0, y[0] * nch + i, 0, 0)),
            pl.BlockSpec((B, CHUNK, H, D), lambda i, y: (0, y[0] * nch + i, 0, 0)),
        ],
        out_specs=pl.BlockSpec((B, SQ, HD), lambda i, y: (0, 0, 0)),
        scratch_shapes=[
            pltpu.VMEM((H, B, CHUNK, D), jnp.float32),
            pltpu.VMEM((H, B, CHUNK, D), jnp.float32),
            pltpu.SemaphoreType.DMA((2, H)),
            pltpu.VMEM((B, SQ, HD), jnp.float32),
            pltpu.VMEM((B, SQ, H), jnp.float32),
            pltpu.VMEM((B, SQ, H), jnp.float32),
            pltpu.VMEM((B, SQ, HD), jnp.bfloat16),
            pltpu.VMEM((B, SQ, HD), jnp.bfloat16),
            pltpu.VMEM((B, SQ, H), jnp.float32),
            pltpu.VMEM((B, SQ, H), jnp.float32),
            pltpu.VMEM((B, SQ, HD), jnp.bfloat16),
            pltpu.VMEM((B, SQ, HD), jnp.bfloat16),
            pltpu.VMEM((B, SQ, H), jnp.float32),
            pltpu.VMEM((B, SQ, H), jnp.float32),
            pltpu.SemaphoreType.DMA((4,)),
            pltpu.SemaphoreType.DMA((4,)),
        ],
    )

    y_idx = jnp.full((1,), lax.axis_index("y"), dtype=jnp.int32)
    out = pl.pallas_call(
        body,
        grid_spec=grid_spec,
        out_shape=jax.ShapeDtypeStruct((B, SQ, HD), jnp.float32),
        compiler_params=pltpu.CompilerParams(
            dimension_semantics=("arbitrary",),
            collective_id=0,
            vmem_limit_bytes=64 * 1024 * 1024,
        ),
    )(y_idx, Q.reshape(B, SQ, HD), K, V)
    return out.reshape(B, SQ, H, D)
